# baseline (speedup 1.0000x reference)
import jax
import jax.numpy as jnp
from jax import lax
from jax.experimental import pallas as pl
from jax.experimental.pallas import tpu as pltpu
from jax.experimental.pallas import tpu_sc as plsc

NN = 10000
NNZ = 320000
D = 128
NV = 2
NS = 16
IB = 128
TPR = 160
IBLK = 40
NBLK = TPR // IBLK
PAD = NS * TPR * IB - NNZ
NTAB = 20480
TRASH_D = 64
ACC_R = NN + TRASH_D
F32 = jnp.float32
SUB = 4
SB = IB // SUB


def _zero_rows(rows_ref, nrows):
    z16 = jnp.zeros((16,), F32)

    def body(i, _):
        r = i // (D // 16)
        k = i % (D // 16)
        rows_ref[r, pl.ds(k * 16, 16)] = z16
        return 0

    lax.fori_loop(0, nrows * (D // 16), body, 0)


def _fill_vec(ref, n, val):
    v16 = jnp.full((16,), val, F32)

    def body(i, _):
        ref[pl.ds(i * 16, 16)] = v16
        return 0

    lax.fori_loop(0, n // 16, body, 0)


def _recip_writeout(deg_sh, sh_off, out_ref, out_off, rbuf, n):
    pltpu.sync_copy(deg_sh.at[pl.ds(sh_off, n)], rbuf.at[pl.ds(0, n)])

    def body(i, _):
        d = rbuf[pl.ds(i * 16, 16)]
        rbuf[pl.ds(i * 16, 16)] = 1.0 / jnp.maximum(d, 1.0)
        return 0

    lax.fori_loop(0, n // 16, body, 0)
    pltpu.sync_copy(rbuf.at[pl.ds(0, n)], out_ref.at[pl.ds(out_off, n)])


def _make_seg_sum(mode):
    with_deg = mode == "deg"
    mesh = plsc.VectorSubcoreMesh(
        core_axis_name="c", subcore_axis_name="s", num_cores=NV,
        num_subcores=NS)
    out_type = [jax.ShapeDtypeStruct((NTAB, D), F32)]
    scratch = [
        pltpu.VMEM((IBLK, IB), jnp.int32),
        pltpu.VMEM((IBLK, IB), jnp.int32),
        pltpu.VMEM((IB, D), F32),
        pltpu.VMEM((IB, D), F32),
        pltpu.VMEM_SHARED((ACC_R, D), F32),
        pltpu.SemaphoreType.DMA,
        pltpu.SemaphoreType.DMA,
    ]
    if mode == "input":
        scratch += [pltpu.VMEM((640,), F32)]
    if with_deg:
        out_type += [jax.ShapeDtypeStruct((NTAB,), F32),
                     jax.ShapeDtypeStruct((NTAB,), F32)]
        scratch += [
            pltpu.VMEM((IB,), F32),
            pltpu.VMEM((640,), F32),
            pltpu.VMEM_SHARED((ACC_R,), F32),
            pltpu.VMEM_SHARED((NTAB,), F32),
        ]

    def body(table, src3d, dst3d, *refs):
        rec_in = None
        if with_deg:
            (acc_out, rece_out, recv_out, idxs, idxd, rows, rows1, acc_sh,
             sem, sem1, ones, rbuf, dege_sh, degv_sh) = refs
        elif mode == "input":
            (rec_in, acc_out, idxs, idxd, rows, rows1, acc_sh, sem, sem1,
             rbuf) = refs
        else:
            acc_out, idxs, idxd, rows, rows1, acc_sh, sem, sem1 = refs
        c = lax.axis_index("c")
        s = lax.axis_index("s")

        _zero_rows(rows, IB)

        def zero_acc(base, nrows):
            for k in range(nrows // IB):
                pltpu.sync_copy(rows, acc_sh.at[pl.ds(base + k * IB, IB)])
            tail = nrows % IB
            if tail:
                pltpu.sync_copy(rows.at[pl.ds(0, tail)],
                                acc_sh.at[pl.ds(base + (nrows // IB) * IB,
                                                tail)])

        @pl.when(s < NS - 1)
        def _():
            zero_acc(s * 624, 624)

        @pl.when(s == NS - 1)
        def _():
            zero_acc(9360, 640)

        if with_deg:
            _fill_vec(rbuf, 640, 0.0)
            _fill_vec(ones, IB, 1.0)

            @pl.when(s < NS - 1)
            def _():
                pltpu.sync_copy(rbuf, dege_sh.at[pl.ds(s * 640, 640)])
                pltpu.sync_copy(rbuf, degv_sh.at[pl.ds(c * NN + s * 640, 640)])

            @pl.when(s == NS - 1)
            def _():
                pltpu.sync_copy(rbuf.at[pl.ds(0, 400)],
                                dege_sh.at[pl.ds(9600, 400)])
                pltpu.sync_copy(rbuf.at[pl.ds(0, 400)],
                                degv_sh.at[pl.ds(c * NN + 9600, 400)])

        plsc.subcore_barrier()

        w = c * NS + s
        bufs = (rows, rows1)
        sems = (sem, sem1)

        def scatter(j, buf):
            pltpu.sync_copy(buf, acc_sh.at[idxd.at[j]], add=True)
            if with_deg:
                pltpu.sync_copy(ones, dege_sh.at[idxd.at[j]], add=True)
                pltpu.sync_copy(ones, degv_sh.at[idxs.at[j]], add=True)

        def fire(j, buf, sm):
            for q in range(SUB):
                pltpu.async_copy(table.at[idxs.at[j, pl.ds(q * SB, SB)]],
                                 buf.at[pl.ds(q * SB, SB)], sm)

        def pair(k, _):
            for h in range(2):
                j = 2 * k + h
                buf, sm = bufs[h], sems[h]
                nbuf, nsm = bufs[1 - h], sems[1 - h]
                pltpu.make_async_copy(table.at[idxs.at[j]], buf, sm).wait()
                if h == 0:
                    fire(j + 1, nbuf, nsm)
                else:
                    @pl.when(k < IBLK // 2 - 1)
                    def _():
                        fire(j + 1, nbuf, nsm)
                scatter(j, buf)
            return 0

        for blk in range(NBLK):
            pltpu.sync_copy(src3d.at[w, pl.ds(blk * IBLK, IBLK)], idxs)
            pltpu.sync_copy(dst3d.at[w, pl.ds(blk * IBLK, IBLK)], idxd)
            fire(0, rows, sem)
            lax.fori_loop(0, IBLK // 2, pair, 0)

        plsc.subcore_barrier()

        def writeout(base, nrows):
            if mode == "deg":
                pltpu.sync_copy(dege_sh.at[pl.ds(base, nrows)],
                                rbuf.at[pl.ds(0, nrows)])

                def rec(i, _):
                    d = rbuf[pl.ds(i * 16, 16)]
                    rbuf[pl.ds(i * 16, 16)] = 1.0 / jnp.maximum(d, 1.0)
                    return 0

                lax.fori_loop(0, nrows // 16, rec, 0)
            elif mode == "input":
                pltpu.sync_copy(rec_in.at[pl.ds(c * NN + base, nrows)],
                                rbuf.at[pl.ds(0, nrows)])

            def chunk_out(off, cn):
                dst = rows.at[pl.ds(0, cn)] if cn < IB else rows
                pltpu.sync_copy(acc_sh.at[pl.ds(base + off, cn)], dst)
                if mode != "raw":
                    def sgrp(g, _):
                        rv = rbuf[pl.ds(off + g * 16, 16)]
                        for r2 in range(16):
                            r = g * 16 + r2
                            v = rv[r2]
                            for kk in range(D // 16):
                                rows[r, pl.ds(kk * 16, 16)] = (
                                    rows[r, pl.ds(kk * 16, 16)] * v)
                        return 0

                    lax.fori_loop(0, cn // 16, sgrp, 0)
                pltpu.sync_copy(dst,
                                acc_out.at[pl.ds(c * NN + base + off, cn)])

            for k in range(nrows // IB):
                chunk_out(k * IB, IB)
            tail = nrows % IB
            if tail:
                chunk_out((nrows // IB) * IB, tail)

        @pl.when(s < NS - 1)
        def _():
            writeout(s * 624, 624)

        @pl.when(s == NS - 1)
        def _():
            writeout(9360, 640)

        if with_deg:
            @pl.when(s < NS - 1)
            def _():
                _recip_writeout(dege_sh, s * 640, rece_out,
                                c * NN + s * 640, rbuf, 640)
                _recip_writeout(degv_sh, c * NN + s * 640, recv_out,
                                c * NN + s * 640, rbuf, 640)

            @pl.when(s == NS - 1)
            def _():
                _recip_writeout(dege_sh, 9600, rece_out,
                                c * NN + 9600, rbuf, 400)
                _recip_writeout(degv_sh, c * NN + 9600, recv_out,
                                c * NN + 9600, rbuf, 400)

    return pl.kernel(body, out_type=out_type, mesh=mesh,
                     scratch_types=scratch)


_seg_sum_deg = _make_seg_sum("deg")
_seg_sum_scaled = _make_seg_sum("input")
_seg_sum = _make_seg_sum("raw")


_BR = 1024
_GRID = NTAB // _BR

_row_spec = pl.BlockSpec((_BR, D), lambda i: (i, 0))
_rec_spec = pl.BlockSpec((_BR, 1), lambda i: (i, 0))
_w_spec = pl.BlockSpec((D, D), lambda i: (0, 0))
_b_spec = pl.BlockSpec((1, D), lambda i: (0, 0))


def _dot(a, w):
    return jax.lax.dot_general(a, w, (((1,), (0,)), ((), ())),
                               precision=jax.lax.Precision.HIGHEST,
                               preferred_element_type=F32)


def _mm_bias(x, w, b):
    def body(x_ref, w_ref, b_ref, o_ref):
        o_ref[...] = _dot(x_ref[...], w_ref[...]) + b_ref[...]

    return pl.pallas_call(
        body, grid=(_GRID,),
        in_specs=[_row_spec, _w_spec, _b_spec],
        out_specs=_row_spec,
        out_shape=jax.ShapeDtypeStruct((NTAB, D), F32),
    )(x, w, b.reshape(1, D))


def _relu_scale_mm(a, r, w, b):
    def body(a_ref, r_ref, w_ref, b_ref, o_ref):
        n = jnp.maximum(a_ref[...] * r_ref[...], 0.0)
        o_ref[...] = _dot(n, w_ref[...]) + b_ref[...]

    return pl.pallas_call(
        body, grid=(_GRID,),
        in_specs=[_row_spec, _rec_spec, _w_spec, _b_spec],
        out_specs=_row_spec,
        out_shape=jax.ShapeDtypeStruct((NTAB, D), F32),
    )(a, r.reshape(NTAB, 1), w, b.reshape(1, D))


def _final(a, r, wp1, bp1, wp2, bp2):
    def body(a_ref, r_ref, w1_ref, b1_ref, w2_ref, b2_ref, n_ref, p_ref):
        n = jnp.maximum(a_ref[...] * r_ref[...], 0.0)
        n_ref[...] = n
        h = _dot(n, w1_ref[...]) + b1_ref[...]
        h = jnp.where(h > 0.0, h, jnp.exp(jnp.minimum(h, 0.0)) - 1.0)
        p_ref[...] = _dot(h, w2_ref[...]) + b2_ref[...]

    return pl.pallas_call(
        body, grid=(_GRID,),
        in_specs=[_row_spec, _rec_spec, _w_spec, _b_spec, _w_spec, _b_spec],
        out_specs=[_row_spec, _row_spec],
        out_shape=[jax.ShapeDtypeStruct((NTAB, D), F32),
                   jax.ShapeDtypeStruct((NTAB, D), F32)],
    )(a, r.reshape(NTAB, 1), wp1, bp1.reshape(1, D),
      wp2, bp2.reshape(1, D))


def _pack_idx(i0, i1):
    both = jnp.stack([i0, i1])
    return both.reshape(NV * NS, TPR, IB)


def kernel(x0, x1, inc0, inc1, W0, b0, W1, b1, Wp1, bp1, Wp2, bp2):
    xs = jnp.concatenate(
        [x0, x1, jnp.zeros((NTAB - NV * NN, D), F32)], axis=0)
    v0, e0 = inc0[0], inc0[1]
    v1, e1 = inc1[0], inc1[1]
    pad_src = NV * NN + (jnp.arange(PAD, dtype=jnp.int32) % (NTAB - NV * NN))
    pad_dst = NN + (jnp.arange(PAD, dtype=jnp.int32) % TRASH_D)
    src_v = _pack_idx(jnp.concatenate([v0, pad_src]),
                      jnp.concatenate([v1 + NN, pad_src]))
    dst_e = _pack_idx(jnp.concatenate([e0, pad_dst]),
                      jnp.concatenate([e1, pad_dst]))
    src_e = _pack_idx(jnp.concatenate([e0, pad_src]),
                      jnp.concatenate([e1 + NN, pad_src]))
    dst_v = _pack_idx(jnp.concatenate([v0, pad_dst]),
                      jnp.concatenate([v1, pad_dst]))

    x_lin = _mm_bias(xs, W0, b0)
    y1, rec_e, rec_v = _seg_sum_deg(x_lin, src_v, dst_e)
    b1_raw, = _seg_sum(y1, src_e, dst_v)
    x2 = _relu_scale_mm(b1_raw, rec_v, W1, b1)
    y2, = _seg_sum_scaled(x2, src_v, dst_e, rec_e)
    b2_raw, = _seg_sum(y2, src_e, dst_v)
    n_out, p_out = _final(b2_raw, rec_v, Wp1, bp1, Wp2, bp2)
    return (n_out[:NN], n_out[NN:NV * NN], p_out[:NN], p_out[NN:NV * NN])


# speedup vs baseline: 1.1872x; 1.1872x over previous
import jax
import jax.numpy as jnp
from jax import lax
from jax.experimental import pallas as pl
from jax.experimental.pallas import tpu as pltpu
from jax.experimental.pallas import tpu_sc as plsc

NN = 10000
NNZ = 320000
D = 128
NV = 2
NS = 16
IB = 64
TPR = 320
IBLK = 40
NBLK = TPR // IBLK
NBUF = 4
PAD = NS * TPR * IB - NNZ
NTAB = 20480
TRASH_D = 64
ACC_R = NN + TRASH_D
F32 = jnp.float32


def _zero_rows(rows_ref, nrows):
    z16 = jnp.zeros((16,), F32)

    def body(i, _):
        r = i // (D // 16)
        k = i % (D // 16)
        rows_ref[r, pl.ds(k * 16, 16)] = z16
        return 0

    lax.fori_loop(0, nrows * (D // 16), body, 0)


def _fill_vec(ref, n, val):
    v16 = jnp.full((16,), val, F32)

    def body(i, _):
        ref[pl.ds(i * 16, 16)] = v16
        return 0

    lax.fori_loop(0, n // 16, body, 0)


def _recip_writeout(deg_sh, sh_off, out_ref, out_off, rbuf, n):
    pltpu.sync_copy(deg_sh.at[pl.ds(sh_off, n)], rbuf.at[pl.ds(0, n)])

    def body(i, _):
        d = rbuf[pl.ds(i * 16, 16)]
        rbuf[pl.ds(i * 16, 16)] = 1.0 / jnp.maximum(d, 1.0)
        return 0

    lax.fori_loop(0, n // 16, body, 0)
    pltpu.sync_copy(rbuf.at[pl.ds(0, n)], out_ref.at[pl.ds(out_off, n)])


def _make_seg_sum(mode):
    with_deg = mode == "deg"
    mesh = plsc.VectorSubcoreMesh(
        core_axis_name="c", subcore_axis_name="s", num_cores=NV,
        num_subcores=NS)
    out_type = [jax.ShapeDtypeStruct((NTAB, D), F32)]
    scratch = [
        pltpu.VMEM((IBLK, IB), jnp.int32),
        pltpu.VMEM((IBLK, IB), jnp.int32),
    ] + [pltpu.VMEM((IB, D), F32) for _ in range(NBUF)] + [
        pltpu.VMEM_SHARED((ACC_R, D), F32),
    ] + [pltpu.SemaphoreType.DMA for _ in range(NBUF)]
    if mode == "input":
        scratch += [pltpu.VMEM((640,), F32)]
    if with_deg:
        out_type += [jax.ShapeDtypeStruct((NTAB,), F32),
                     jax.ShapeDtypeStruct((NTAB,), F32)]
        scratch += [
            pltpu.VMEM((IB,), F32),
            pltpu.VMEM((640,), F32),
            pltpu.VMEM_SHARED((ACC_R,), F32),
            pltpu.VMEM_SHARED((NTAB,), F32),
        ]

    def body(table, src3d, dst3d, *refs):
        rec_in = None
        if with_deg:
            (acc_out, rece_out, recv_out, idxs, idxd, *rest) = refs
            bufs, rest = rest[:NBUF], rest[NBUF:]
            acc_sh, *rest = rest
            sems, rest = rest[:NBUF], rest[NBUF:]
            ones, rbuf, dege_sh, degv_sh = rest
        elif mode == "input":
            (rec_in, acc_out, idxs, idxd, *rest) = refs
            bufs, rest = rest[:NBUF], rest[NBUF:]
            acc_sh, *rest = rest
            sems, rest = rest[:NBUF], rest[NBUF:]
            rbuf, = rest
        else:
            (acc_out, idxs, idxd, *rest) = refs
            bufs, rest = rest[:NBUF], rest[NBUF:]
            acc_sh, *rest = rest
            sems = rest[:NBUF]
        rows = bufs[0]
        c = lax.axis_index("c")
        s = lax.axis_index("s")

        _zero_rows(rows, IB)

        def zero_acc(base, nrows):
            for k in range(nrows // IB):
                pltpu.sync_copy(rows, acc_sh.at[pl.ds(base + k * IB, IB)])
            tail = nrows % IB
            if tail:
                pltpu.sync_copy(rows.at[pl.ds(0, tail)],
                                acc_sh.at[pl.ds(base + (nrows // IB) * IB,
                                                tail)])

        @pl.when(s < NS - 1)
        def _():
            zero_acc(s * 624, 624)

        @pl.when(s == NS - 1)
        def _():
            zero_acc(9360, 640)

        if with_deg:
            _fill_vec(rbuf, 640, 0.0)
            _fill_vec(ones, IB, 1.0)

            @pl.when(s < NS - 1)
            def _():
                pltpu.sync_copy(rbuf, dege_sh.at[pl.ds(s * 640, 640)])
                pltpu.sync_copy(rbuf, degv_sh.at[pl.ds(c * NN + s * 640, 640)])

            @pl.when(s == NS - 1)
            def _():
                pltpu.sync_copy(rbuf.at[pl.ds(0, 400)],
                                dege_sh.at[pl.ds(9600, 400)])
                pltpu.sync_copy(rbuf.at[pl.ds(0, 400)],
                                degv_sh.at[pl.ds(c * NN + 9600, 400)])

        plsc.subcore_barrier()

        w = c * NS + s

        def scatter(j, buf):
            pltpu.sync_copy(buf, acc_sh.at[idxd.at[j]], add=True)
            if with_deg:
                pltpu.sync_copy(ones, dege_sh.at[idxd.at[j]], add=True)
                pltpu.sync_copy(ones, degv_sh.at[idxs.at[j]], add=True)

        def quad(k, _):
            for h in range(NBUF):
                j = NBUF * k + h
                pltpu.make_async_copy(table.at[idxs.at[j]], bufs[h],
                                      sems[h]).wait()
                scatter(j, bufs[h])

                @pl.when(k < IBLK // NBUF - 1)
                def _():
                    pltpu.async_copy(table.at[idxs.at[j + NBUF]], bufs[h],
                                     sems[h])
            return 0

        for blk in range(NBLK):
            pltpu.sync_copy(src3d.at[w, pl.ds(blk * IBLK, IBLK)], idxs)
            pltpu.sync_copy(dst3d.at[w, pl.ds(blk * IBLK, IBLK)], idxd)
            for h in range(NBUF):
                pltpu.async_copy(table.at[idxs.at[h]], bufs[h], sems[h])
            lax.fori_loop(0, IBLK // NBUF, quad, 0)

        plsc.subcore_barrier()

        def writeout(base, nrows):
            if mode == "deg":
                pltpu.sync_copy(dege_sh.at[pl.ds(base, nrows)],
                                rbuf.at[pl.ds(0, nrows)])

                def rec(i, _):
                    d = rbuf[pl.ds(i * 16, 16)]
                    rbuf[pl.ds(i * 16, 16)] = 1.0 / jnp.maximum(d, 1.0)
                    return 0

                lax.fori_loop(0, nrows // 16, rec, 0)
            elif mode == "input":
                pltpu.sync_copy(rec_in.at[pl.ds(c * NN + base, nrows)],
                                rbuf.at[pl.ds(0, nrows)])

            def chunk_out(off, cn):
                dst = rows.at[pl.ds(0, cn)] if cn < IB else rows
                pltpu.sync_copy(acc_sh.at[pl.ds(base + off, cn)], dst)
                if mode != "raw":
                    def sgrp(g, _):
                        rv = rbuf[pl.ds(off + g * 16, 16)]
                        for r2 in range(16):
                            r = g * 16 + r2
                            v = rv[r2]
                            for kk in range(D // 16):
                                rows[r, pl.ds(kk * 16, 16)] = (
                                    rows[r, pl.ds(kk * 16, 16)] * v)
                        return 0

                    lax.fori_loop(0, cn // 16, sgrp, 0)
                pltpu.sync_copy(dst,
                                acc_out.at[pl.ds(c * NN + base + off, cn)])

            for k in range(nrows // IB):
                chunk_out(k * IB, IB)
            tail = nrows % IB
            if tail:
                chunk_out((nrows // IB) * IB, tail)

        @pl.when(s < NS - 1)
        def _():
            writeout(s * 624, 624)

        @pl.when(s == NS - 1)
        def _():
            writeout(9360, 640)

        if with_deg:
            @pl.when(s < NS - 1)
            def _():
                _recip_writeout(dege_sh, s * 640, rece_out,
                                c * NN + s * 640, rbuf, 640)
                _recip_writeout(degv_sh, c * NN + s * 640, recv_out,
                                c * NN + s * 640, rbuf, 640)

            @pl.when(s == NS - 1)
            def _():
                _recip_writeout(dege_sh, 9600, rece_out,
                                c * NN + 9600, rbuf, 400)
                _recip_writeout(degv_sh, c * NN + 9600, recv_out,
                                c * NN + 9600, rbuf, 400)

    return pl.kernel(body, out_type=out_type, mesh=mesh,
                     scratch_types=scratch)


_seg_sum_deg = _make_seg_sum("deg")
_seg_sum_scaled = _make_seg_sum("input")
_seg_sum = _make_seg_sum("raw")


_BR = 1024
_GRID = NTAB // _BR

_row_spec = pl.BlockSpec((_BR, D), lambda i: (i, 0))
_rec_spec = pl.BlockSpec((_BR, 1), lambda i: (i, 0))
_w_spec = pl.BlockSpec((D, D), lambda i: (0, 0))
_b_spec = pl.BlockSpec((1, D), lambda i: (0, 0))


def _dot(a, w):
    return jax.lax.dot_general(a, w, (((1,), (0,)), ((), ())),
                               precision=jax.lax.Precision.HIGHEST,
                               preferred_element_type=F32)


def _mm_bias(x, w, b):
    def body(x_ref, w_ref, b_ref, o_ref):
        o_ref[...] = _dot(x_ref[...], w_ref[...]) + b_ref[...]

    return pl.pallas_call(
        body, grid=(_GRID,),
        in_specs=[_row_spec, _w_spec, _b_spec],
        out_specs=_row_spec,
        out_shape=jax.ShapeDtypeStruct((NTAB, D), F32),
    )(x, w, b.reshape(1, D))


def _relu_scale_mm(a, r, w, b):
    def body(a_ref, r_ref, w_ref, b_ref, o_ref):
        n = jnp.maximum(a_ref[...] * r_ref[...], 0.0)
        o_ref[...] = _dot(n, w_ref[...]) + b_ref[...]

    return pl.pallas_call(
        body, grid=(_GRID,),
        in_specs=[_row_spec, _rec_spec, _w_spec, _b_spec],
        out_specs=_row_spec,
        out_shape=jax.ShapeDtypeStruct((NTAB, D), F32),
    )(a, r.reshape(NTAB, 1), w, b.reshape(1, D))


def _final(a, r, wp1, bp1, wp2, bp2):
    def body(a_ref, r_ref, w1_ref, b1_ref, w2_ref, b2_ref, n_ref, p_ref):
        n = jnp.maximum(a_ref[...] * r_ref[...], 0.0)
        n_ref[...] = n
        h = _dot(n, w1_ref[...]) + b1_ref[...]
        h = jnp.where(h > 0.0, h, jnp.exp(jnp.minimum(h, 0.0)) - 1.0)
        p_ref[...] = _dot(h, w2_ref[...]) + b2_ref[...]

    return pl.pallas_call(
        body, grid=(_GRID,),
        in_specs=[_row_spec, _rec_spec, _w_spec, _b_spec, _w_spec, _b_spec],
        out_specs=[_row_spec, _row_spec],
        out_shape=[jax.ShapeDtypeStruct((NTAB, D), F32),
                   jax.ShapeDtypeStruct((NTAB, D), F32)],
    )(a, r.reshape(NTAB, 1), wp1, bp1.reshape(1, D),
      wp2, bp2.reshape(1, D))


def _pack_idx(i0, i1):
    both = jnp.stack([i0, i1])
    return both.reshape(NV * NS, TPR, IB)


def kernel(x0, x1, inc0, inc1, W0, b0, W1, b1, Wp1, bp1, Wp2, bp2):
    xs = jnp.concatenate(
        [x0, x1, jnp.zeros((NTAB - NV * NN, D), F32)], axis=0)
    v0, e0 = inc0[0], inc0[1]
    v1, e1 = inc1[0], inc1[1]
    pad_src = NV * NN + (jnp.arange(PAD, dtype=jnp.int32) % (NTAB - NV * NN))
    pad_dst = NN + (jnp.arange(PAD, dtype=jnp.int32) % TRASH_D)
    src_v = _pack_idx(jnp.concatenate([v0, pad_src]),
                      jnp.concatenate([v1 + NN, pad_src]))
    dst_e = _pack_idx(jnp.concatenate([e0, pad_dst]),
                      jnp.concatenate([e1, pad_dst]))
    src_e = _pack_idx(jnp.concatenate([e0, pad_src]),
                      jnp.concatenate([e1 + NN, pad_src]))
    dst_v = _pack_idx(jnp.concatenate([v0, pad_dst]),
                      jnp.concatenate([v1, pad_dst]))

    x_lin = _mm_bias(xs, W0, b0)
    y1, rec_e, rec_v = _seg_sum_deg(x_lin, src_v, dst_e)
    b1_raw, = _seg_sum(y1, src_e, dst_v)
    x2 = _relu_scale_mm(b1_raw, rec_v, W1, b1)
    y2, = _seg_sum_scaled(x2, src_v, dst_e, rec_e)
    b2_raw, = _seg_sum(y2, src_e, dst_v)
    n_out, p_out = _final(b2_raw, rec_v, Wp1, bp1, Wp2, bp2)
    return (n_out[:NN], n_out[NN:NV * NN], p_out[:NN], p_out[NN:NV * NN])


# speedup vs baseline: 1.3032x; 1.0977x over previous
import jax
import jax.numpy as jnp
from jax import lax
from jax.experimental import pallas as pl
from jax.experimental.pallas import tpu as pltpu
from jax.experimental.pallas import tpu_sc as plsc

NN = 10000
NNZ = 320000
D = 128
NV = 2
NS = 16
IB = 64
TPR = 320
IBLK = 16
NBLK = TPR // IBLK
NBUF = 4
PAD = NS * TPR * IB - NNZ
NTAB = 20480
TRASH_D = 64
ACC_R = NN + TRASH_D
F32 = jnp.float32


def _zero_rows(rows_ref, nrows):
    z16 = jnp.zeros((16,), F32)

    def body(i, _):
        r = i // (D // 16)
        k = i % (D // 16)
        rows_ref[r, pl.ds(k * 16, 16)] = z16
        return 0

    lax.fori_loop(0, nrows * (D // 16), body, 0)


def _fill_vec(ref, n, val):
    v16 = jnp.full((16,), val, F32)

    def body(i, _):
        ref[pl.ds(i * 16, 16)] = v16
        return 0

    lax.fori_loop(0, n // 16, body, 0)


def _recip_writeout(deg_sh, sh_off, out_ref, out_off, rbuf, n):
    pltpu.sync_copy(deg_sh.at[pl.ds(sh_off, n)], rbuf.at[pl.ds(0, n)])

    def body(i, _):
        d = rbuf[pl.ds(i * 16, 16)]
        rbuf[pl.ds(i * 16, 16)] = 1.0 / jnp.maximum(d, 1.0)
        return 0

    lax.fori_loop(0, n // 16, body, 0)
    pltpu.sync_copy(rbuf.at[pl.ds(0, n)], out_ref.at[pl.ds(out_off, n)])


def _make_seg_sum(mode):
    with_deg = mode == "deg"
    mesh = plsc.VectorSubcoreMesh(
        core_axis_name="c", subcore_axis_name="s", num_cores=NV,
        num_subcores=NS)
    out_type = [jax.ShapeDtypeStruct((NTAB, D), F32)]
    scratch = [
        pltpu.VMEM((IBLK, IB), jnp.int32),
        pltpu.VMEM((IBLK, IB), jnp.int32),
        pltpu.VMEM((IBLK, IB), jnp.int32),
        pltpu.VMEM((IBLK, IB), jnp.int32),
    ] + [pltpu.VMEM((IB, D), F32) for _ in range(NBUF)] + [
        pltpu.VMEM_SHARED((ACC_R, D), F32),
    ] + [pltpu.SemaphoreType.DMA for _ in range(NBUF + 2)]
    if mode == "input":
        scratch += [pltpu.VMEM((640,), F32)]
    if with_deg:
        out_type += [jax.ShapeDtypeStruct((NTAB,), F32),
                     jax.ShapeDtypeStruct((NTAB,), F32)]
        scratch += [
            pltpu.VMEM((IB,), F32),
            pltpu.VMEM((640,), F32),
            pltpu.VMEM_SHARED((ACC_R,), F32),
            pltpu.VMEM_SHARED((NTAB,), F32),
        ]

    def body(table, src3d, dst3d, *refs):
        rec_in = None
        if with_deg:
            (acc_out, rece_out, recv_out, *rest) = refs
        elif mode == "input":
            (rec_in, acc_out, *rest) = refs
        else:
            (acc_out, *rest) = refs
        ipairs = ((rest[0], rest[1]), (rest[2], rest[3]))
        rest = rest[4:]
        bufs, rest = rest[:NBUF], rest[NBUF:]
        acc_sh, *rest = rest
        sems, rest = rest[:NBUF], rest[NBUF:]
        isems, rest = rest[:2], rest[2:]
        if with_deg:
            ones, rbuf, dege_sh, degv_sh = rest
        elif mode == "input":
            rbuf, = rest
        rows = bufs[0]
        c = lax.axis_index("c")
        s = lax.axis_index("s")

        _zero_rows(rows, IB)

        def zero_acc(base, nrows):
            for k in range(nrows // IB):
                pltpu.sync_copy(rows, acc_sh.at[pl.ds(base + k * IB, IB)])
            tail = nrows % IB
            if tail:
                pltpu.sync_copy(rows.at[pl.ds(0, tail)],
                                acc_sh.at[pl.ds(base + (nrows // IB) * IB,
                                                tail)])

        @pl.when(s < NS - 1)
        def _():
            zero_acc(s * 624, 624)

        @pl.when(s == NS - 1)
        def _():
            zero_acc(9360, 640)

        if with_deg:
            _fill_vec(rbuf, 640, 0.0)
            _fill_vec(ones, IB, 1.0)

            @pl.when(s < NS - 1)
            def _():
                pltpu.sync_copy(rbuf, dege_sh.at[pl.ds(s * 640, 640)])
                pltpu.sync_copy(rbuf, degv_sh.at[pl.ds(c * NN + s * 640, 640)])

            @pl.when(s == NS - 1)
            def _():
                pltpu.sync_copy(rbuf.at[pl.ds(0, 400)],
                                dege_sh.at[pl.ds(9600, 400)])
                pltpu.sync_copy(rbuf.at[pl.ds(0, 400)],
                                degv_sh.at[pl.ds(c * NN + 9600, 400)])

        plsc.subcore_barrier()

        w = c * NS + s

        def islices(b):
            off = pl.multiple_of(b * IBLK, 8)
            return (src3d.at[w, pl.ds(off, IBLK)],
                    dst3d.at[w, pl.ds(off, IBLK)])

        def idx_load(b, slot, sync):
            slcs = islices(b)
            for src, dst in zip(slcs, ipairs[slot]):
                if sync:
                    pltpu.sync_copy(src, dst)
                else:
                    pltpu.async_copy(src, dst, isems[slot])

        def idx_wait(b, slot):
            slcs = islices(b)
            for src, dst in zip(slcs, ipairs[slot]):
                pltpu.make_async_copy(src, dst, isems[slot]).wait()

        def proc(pair, j, h, nxt):
            ps, pd = pair
            pltpu.make_async_copy(table.at[ps.at[j]], bufs[h],
                                  sems[h]).wait()
            pltpu.sync_copy(bufs[h], acc_sh.at[pd.at[j]], add=True)
            if with_deg:
                pltpu.sync_copy(ones, dege_sh.at[pd.at[j]], add=True)
                pltpu.sync_copy(ones, degv_sh.at[ps.at[j]], add=True)
            if nxt is not None:
                nps, nj = nxt
                pltpu.async_copy(table.at[nps.at[nj]], bufs[h], sems[h])

        NQ = IBLK // NBUF

        def run_block(b, slot, last):
            pair = ipairs[slot]
            nslot = (slot + 1) % 2
            if not last:
                idx_load(b + 1, nslot, False)

            def quad(k, _):
                for h in range(NBUF):
                    j = NBUF * k + h
                    proc(pair, j, h, (pair[0], j + NBUF))
                return 0

            lax.fori_loop(0, NQ - 1, quad, 0)
            if not last:
                idx_wait(b + 1, nslot)
                for h in range(NBUF):
                    proc(pair, IBLK - NBUF + h, h, (ipairs[nslot][0], h))
            else:
                for h in range(NBUF):
                    proc(pair, IBLK - NBUF + h, h, None)

        idx_load(0, 0, True)
        for h in range(NBUF):
            pltpu.async_copy(table.at[ipairs[0][0].at[h]], bufs[h], sems[h])
        run_block(0, 0, False)

        def gbody(g, _):
            b = 1 + 2 * g
            run_block(b, 1, False)
            run_block(b + 1, 0, False)
            return 0

        lax.fori_loop(0, (NBLK - 2) // 2, gbody, 0)
        run_block(NBLK - 1, 1, True)

        plsc.subcore_barrier()

        def writeout(base, nrows):
            if mode == "deg":
                pltpu.sync_copy(dege_sh.at[pl.ds(base, nrows)],
                                rbuf.at[pl.ds(0, nrows)])

                def rec(i, _):
                    d = rbuf[pl.ds(i * 16, 16)]
                    rbuf[pl.ds(i * 16, 16)] = 1.0 / jnp.maximum(d, 1.0)
                    return 0

                lax.fori_loop(0, nrows // 16, rec, 0)
            elif mode == "input":
                pltpu.sync_copy(rec_in.at[pl.ds(c * NN + base, nrows)],
                                rbuf.at[pl.ds(0, nrows)])

            def chunk_out(off, cn):
                dst = rows.at[pl.ds(0, cn)] if cn < IB else rows
                pltpu.sync_copy(acc_sh.at[pl.ds(base + off, cn)], dst)
                if mode != "raw":
                    def sgrp(g, _):
                        rv = rbuf[pl.ds(off + g * 16, 16)]
                        for r2 in range(16):
                            r = g * 16 + r2
                            v = rv[r2]
                            for kk in range(D // 16):
                                rows[r, pl.ds(kk * 16, 16)] = (
                                    rows[r, pl.ds(kk * 16, 16)] * v)
                        return 0

                    lax.fori_loop(0, cn // 16, sgrp, 0)
                pltpu.sync_copy(dst,
                                acc_out.at[pl.ds(c * NN + base + off, cn)])

            for k in range(nrows // IB):
                chunk_out(k * IB, IB)
            tail = nrows % IB
            if tail:
                chunk_out((nrows // IB) * IB, tail)

        @pl.when(s < NS - 1)
        def _():
            writeout(s * 624, 624)

        @pl.when(s == NS - 1)
        def _():
            writeout(9360, 640)

        if with_deg:
            @pl.when(s < NS - 1)
            def _():
                _recip_writeout(dege_sh, s * 640, rece_out,
                                c * NN + s * 640, rbuf, 640)
                _recip_writeout(degv_sh, c * NN + s * 640, recv_out,
                                c * NN + s * 640, rbuf, 640)

            @pl.when(s == NS - 1)
            def _():
                _recip_writeout(dege_sh, 9600, rece_out,
                                c * NN + 9600, rbuf, 400)
                _recip_writeout(degv_sh, c * NN + 9600, recv_out,
                                c * NN + 9600, rbuf, 400)

    return pl.kernel(body, out_type=out_type, mesh=mesh,
                     scratch_types=scratch)


_seg_sum_deg = _make_seg_sum("deg")
_seg_sum_scaled = _make_seg_sum("input")
_seg_sum = _make_seg_sum("raw")


_BR = 1024
_GRID = NTAB // _BR

_row_spec = pl.BlockSpec((_BR, D), lambda i: (i, 0))
_rec_spec = pl.BlockSpec((_BR, 1), lambda i: (i, 0))
_w_spec = pl.BlockSpec((D, D), lambda i: (0, 0))
_b_spec = pl.BlockSpec((1, D), lambda i: (0, 0))


def _dot(a, w):
    return jax.lax.dot_general(a, w, (((1,), (0,)), ((), ())),
                               precision=jax.lax.Precision.HIGHEST,
                               preferred_element_type=F32)


def _mm_bias(x, w, b):
    def body(x_ref, w_ref, b_ref, o_ref):
        o_ref[...] = _dot(x_ref[...], w_ref[...]) + b_ref[...]

    return pl.pallas_call(
        body, grid=(_GRID,),
        in_specs=[_row_spec, _w_spec, _b_spec],
        out_specs=_row_spec,
        out_shape=jax.ShapeDtypeStruct((NTAB, D), F32),
    )(x, w, b.reshape(1, D))


def _relu_scale_mm(a, r, w, b):
    def body(a_ref, r_ref, w_ref, b_ref, o_ref):
        n = jnp.maximum(a_ref[...] * r_ref[...], 0.0)
        o_ref[...] = _dot(n, w_ref[...]) + b_ref[...]

    return pl.pallas_call(
        body, grid=(_GRID,),
        in_specs=[_row_spec, _rec_spec, _w_spec, _b_spec],
        out_specs=_row_spec,
        out_shape=jax.ShapeDtypeStruct((NTAB, D), F32),
    )(a, r.reshape(NTAB, 1), w, b.reshape(1, D))


def _final(a, r, wp1, bp1, wp2, bp2):
    def body(a_ref, r_ref, w1_ref, b1_ref, w2_ref, b2_ref, n_ref, p_ref):
        n = jnp.maximum(a_ref[...] * r_ref[...], 0.0)
        n_ref[...] = n
        h = _dot(n, w1_ref[...]) + b1_ref[...]
        h = jnp.where(h > 0.0, h, jnp.exp(jnp.minimum(h, 0.0)) - 1.0)
        p_ref[...] = _dot(h, w2_ref[...]) + b2_ref[...]

    return pl.pallas_call(
        body, grid=(_GRID,),
        in_specs=[_row_spec, _rec_spec, _w_spec, _b_spec, _w_spec, _b_spec],
        out_specs=[_row_spec, _row_spec],
        out_shape=[jax.ShapeDtypeStruct((NTAB, D), F32),
                   jax.ShapeDtypeStruct((NTAB, D), F32)],
    )(a, r.reshape(NTAB, 1), wp1, bp1.reshape(1, D),
      wp2, bp2.reshape(1, D))


def _pack_idx(i0, i1):
    both = jnp.stack([i0, i1])
    return both.reshape(NV * NS, TPR, IB)


def kernel(x0, x1, inc0, inc1, W0, b0, W1, b1, Wp1, bp1, Wp2, bp2):
    xs = jnp.concatenate(
        [x0, x1, jnp.zeros((NTAB - NV * NN, D), F32)], axis=0)
    v0, e0 = inc0[0], inc0[1]
    v1, e1 = inc1[0], inc1[1]
    pad_src = NV * NN + (jnp.arange(PAD, dtype=jnp.int32) % (NTAB - NV * NN))
    pad_dst = NN + (jnp.arange(PAD, dtype=jnp.int32) % TRASH_D)
    src_v = _pack_idx(jnp.concatenate([v0, pad_src]),
                      jnp.concatenate([v1 + NN, pad_src]))
    dst_e = _pack_idx(jnp.concatenate([e0, pad_dst]),
                      jnp.concatenate([e1, pad_dst]))
    src_e = _pack_idx(jnp.concatenate([e0, pad_src]),
                      jnp.concatenate([e1 + NN, pad_src]))
    dst_v = _pack_idx(jnp.concatenate([v0, pad_dst]),
                      jnp.concatenate([v1, pad_dst]))

    x_lin = _mm_bias(xs, W0, b0)
    y1, rec_e, rec_v = _seg_sum_deg(x_lin, src_v, dst_e)
    b1_raw, = _seg_sum(y1, src_e, dst_v)
    x2 = _relu_scale_mm(b1_raw, rec_v, W1, b1)
    y2, = _seg_sum_scaled(x2, src_v, dst_e, rec_e)
    b2_raw, = _seg_sum(y2, src_e, dst_v)
    n_out, p_out = _final(b2_raw, rec_v, Wp1, bp1, Wp2, bp2)
    return (n_out[:NN], n_out[NN:NV * NN], p_out[:NN], p_out[NN:NV * NN])


# speedup vs baseline: 1.3251x; 1.0169x over previous
import jax
import jax.numpy as jnp
from jax import lax
from jax.experimental import pallas as pl
from jax.experimental.pallas import tpu as pltpu
from jax.experimental.pallas import tpu_sc as plsc

NN = 10000
NNZ = 320000
D = 128
NV = 2
NS = 16
IB = 64
TPR = 320
IBLK = 16
NBLK = TPR // IBLK
NBUF = 4
PAD = NS * TPR * IB - NNZ
NTAB = 20480
TRASH_D = 64
ACC_R = NN + TRASH_D
F32 = jnp.float32


def _zero_rows(rows_ref, nrows):
    z16 = jnp.zeros((16,), F32)

    def body(i, _):
        r = i // (D // 16)
        k = i % (D // 16)
        rows_ref[r, pl.ds(k * 16, 16)] = z16
        return 0

    lax.fori_loop(0, nrows * (D // 16), body, 0)


def _fill_vec(ref, n, val):
    v16 = jnp.full((16,), val, F32)

    def body(i, _):
        ref[pl.ds(i * 16, 16)] = v16
        return 0

    lax.fori_loop(0, n // 16, body, 0)


def _recip_writeout(deg_sh, sh_off, out_ref, out_off, rbuf, n):
    pltpu.sync_copy(deg_sh.at[pl.ds(sh_off, n)], rbuf.at[pl.ds(0, n)])

    def body(i, _):
        d = rbuf[pl.ds(i * 16, 16)]
        rbuf[pl.ds(i * 16, 16)] = 1.0 / jnp.maximum(d, 1.0)
        return 0

    lax.fori_loop(0, n // 16, body, 0)
    pltpu.sync_copy(rbuf.at[pl.ds(0, n)], out_ref.at[pl.ds(out_off, n)])


def _make_seg_sum(mode):
    with_deg = mode == "deg"
    mesh = plsc.VectorSubcoreMesh(
        core_axis_name="c", subcore_axis_name="s", num_cores=NV,
        num_subcores=NS)
    out_type = [jax.ShapeDtypeStruct((NTAB, D), F32)]
    scratch = [
        pltpu.VMEM((IBLK, IB), jnp.int32),
        pltpu.VMEM((IBLK, IB), jnp.int32),
        pltpu.VMEM((IBLK, IB), jnp.int32),
        pltpu.VMEM((IBLK, IB), jnp.int32),
    ] + [pltpu.VMEM((IB, D), F32) for _ in range(NBUF)] + [
        pltpu.VMEM_SHARED((ACC_R, D), F32),
    ] + [pltpu.SemaphoreType.DMA for _ in range(NBUF + 2)]
    if mode == "input":
        scratch += [pltpu.VMEM((640,), F32)]
    if with_deg:
        out_type += [jax.ShapeDtypeStruct((NTAB,), F32),
                     jax.ShapeDtypeStruct((NTAB,), F32)]
        scratch += [
            pltpu.VMEM((IB,), F32),
            pltpu.VMEM((640,), F32),
            pltpu.VMEM_SHARED((ACC_R,), F32),
            pltpu.VMEM_SHARED((NTAB,), F32),
        ]

    def body(table, src3d, dst3d, *refs):
        rec_in = None
        if with_deg:
            (acc_out, rece_out, recv_out, *rest) = refs
        elif mode == "input":
            (rec_in, acc_out, *rest) = refs
        else:
            (acc_out, *rest) = refs
        ipairs = ((rest[0], rest[1]), (rest[2], rest[3]))
        rest = rest[4:]
        bufs, rest = rest[:NBUF], rest[NBUF:]
        acc_sh, *rest = rest
        sems, rest = rest[:NBUF], rest[NBUF:]
        isems, rest = rest[:2], rest[2:]
        if with_deg:
            ones, rbuf, dege_sh, degv_sh = rest
        elif mode == "input":
            rbuf, = rest
        rows = bufs[0]
        c = lax.axis_index("c")
        s = lax.axis_index("s")

        _zero_rows(rows, IB)

        def zero_acc(base, nrows):
            descs = []
            for k in range(nrows // IB):
                descs.append((rows, acc_sh.at[pl.ds(base + k * IB, IB)]))
            tail = nrows % IB
            if tail:
                descs.append((rows.at[pl.ds(0, tail)],
                              acc_sh.at[pl.ds(base + (nrows // IB) * IB,
                                              tail)]))
            for src, dst in descs:
                pltpu.async_copy(src, dst, isems[0])
            for src, dst in descs:
                pltpu.make_async_copy(src, dst, isems[0]).wait()

        @pl.when(s < NS - 1)
        def _():
            zero_acc(s * 624, 624)

        @pl.when(s == NS - 1)
        def _():
            zero_acc(9360, 640)

        if with_deg:
            _fill_vec(rbuf, 640, 0.0)
            _fill_vec(ones, IB, 1.0)

            @pl.when(s < NS - 1)
            def _():
                pltpu.sync_copy(rbuf, dege_sh.at[pl.ds(s * 640, 640)])
                pltpu.sync_copy(rbuf, degv_sh.at[pl.ds(c * NN + s * 640, 640)])

            @pl.when(s == NS - 1)
            def _():
                pltpu.sync_copy(rbuf.at[pl.ds(0, 400)],
                                dege_sh.at[pl.ds(9600, 400)])
                pltpu.sync_copy(rbuf.at[pl.ds(0, 400)],
                                degv_sh.at[pl.ds(c * NN + 9600, 400)])

        plsc.subcore_barrier()

        w = c * NS + s

        def islices(b):
            off = pl.multiple_of(b * IBLK, 8)
            return (src3d.at[w, pl.ds(off, IBLK)],
                    dst3d.at[w, pl.ds(off, IBLK)])

        def idx_load(b, slot, sync):
            slcs = islices(b)
            for src, dst in zip(slcs, ipairs[slot]):
                if sync:
                    pltpu.sync_copy(src, dst)
                else:
                    pltpu.async_copy(src, dst, isems[slot])

        def idx_wait(b, slot):
            slcs = islices(b)
            for src, dst in zip(slcs, ipairs[slot]):
                pltpu.make_async_copy(src, dst, isems[slot]).wait()

        def proc(pair, j, h, nxt):
            ps, pd = pair
            pltpu.make_async_copy(table.at[ps.at[j]], bufs[h],
                                  sems[h]).wait()
            pltpu.sync_copy(bufs[h], acc_sh.at[pd.at[j]], add=True)
            if with_deg:
                pltpu.sync_copy(ones, dege_sh.at[pd.at[j]], add=True)
                pltpu.sync_copy(ones, degv_sh.at[ps.at[j]], add=True)
            if nxt is not None:
                nps, nj = nxt
                pltpu.async_copy(table.at[nps.at[nj]], bufs[h], sems[h])

        NQ = IBLK // NBUF

        def run_block(b, slot, last):
            pair = ipairs[slot]
            nslot = (slot + 1) % 2
            if not last:
                idx_load(b + 1, nslot, False)

            def quad(k, _):
                for h in range(NBUF):
                    j = NBUF * k + h
                    proc(pair, j, h, (pair[0], j + NBUF))
                return 0

            lax.fori_loop(0, NQ - 1, quad, 0)
            if not last:
                idx_wait(b + 1, nslot)
                for h in range(NBUF):
                    proc(pair, IBLK - NBUF + h, h, (ipairs[nslot][0], h))
            else:
                for h in range(NBUF):
                    proc(pair, IBLK - NBUF + h, h, None)

        idx_load(0, 0, True)
        for h in range(NBUF):
            pltpu.async_copy(table.at[ipairs[0][0].at[h]], bufs[h], sems[h])
        run_block(0, 0, False)

        def gbody(g, _):
            b = 1 + 2 * g
            run_block(b, 1, False)
            run_block(b + 1, 0, False)
            return 0

        lax.fori_loop(0, (NBLK - 2) // 2, gbody, 0)
        run_block(NBLK - 1, 1, True)

        plsc.subcore_barrier()

        def writeout(base, nrows):
            if mode == "deg":
                pltpu.sync_copy(dege_sh.at[pl.ds(base, nrows)],
                                rbuf.at[pl.ds(0, nrows)])

                def rec(i, _):
                    d = rbuf[pl.ds(i * 16, 16)]
                    rbuf[pl.ds(i * 16, 16)] = 1.0 / jnp.maximum(d, 1.0)
                    return 0

                lax.fori_loop(0, nrows // 16, rec, 0)
            elif mode == "input":
                pltpu.sync_copy(rec_in.at[pl.ds(c * NN + base, nrows)],
                                rbuf.at[pl.ds(0, nrows)])

            chunks = [(k * IB, IB) for k in range(nrows // IB)]
            if nrows % IB:
                chunks.append(((nrows // IB) * IB, nrows % IB))
            prev = {}
            for i, (off, cn) in enumerate(chunks):
                p = i % 2
                pbuf = bufs[p]
                buf = pbuf.at[pl.ds(0, cn)] if cn < IB else pbuf
                if p in prev:
                    psrc, pdst = prev[p]
                    pltpu.make_async_copy(psrc, pdst, sems[p]).wait()
                pltpu.sync_copy(acc_sh.at[pl.ds(base + off, cn)], buf)
                if mode != "raw":
                    def sgrp(g, _, off=off, pbuf=pbuf):
                        rv = rbuf[pl.ds(off + g * 16, 16)]
                        for r2 in range(16):
                            r = g * 16 + r2
                            v = rv[r2]
                            for kk in range(D // 16):
                                pbuf[r, pl.ds(kk * 16, 16)] = (
                                    pbuf[r, pl.ds(kk * 16, 16)] * v)
                        return 0

                    lax.fori_loop(0, cn // 16, sgrp, 0)
                dst_h = acc_out.at[pl.ds(c * NN + base + off, cn)]
                pltpu.async_copy(buf, dst_h, sems[p])
                prev[p] = (buf, dst_h)
            for p in prev:
                psrc, pdst = prev[p]
                pltpu.make_async_copy(psrc, pdst, sems[p]).wait()

        @pl.when(s < NS - 1)
        def _():
            writeout(s * 624, 624)

        @pl.when(s == NS - 1)
        def _():
            writeout(9360, 640)

        if with_deg:
            @pl.when(s < NS - 1)
            def _():
                _recip_writeout(dege_sh, s * 640, rece_out,
                                c * NN + s * 640, rbuf, 640)
                _recip_writeout(degv_sh, c * NN + s * 640, recv_out,
                                c * NN + s * 640, rbuf, 640)

            @pl.when(s == NS - 1)
            def _():
                _recip_writeout(dege_sh, 9600, rece_out,
                                c * NN + 9600, rbuf, 400)
                _recip_writeout(degv_sh, c * NN + 9600, recv_out,
                                c * NN + 9600, rbuf, 400)

    return pl.kernel(body, out_type=out_type, mesh=mesh,
                     scratch_types=scratch)


_seg_sum_deg = _make_seg_sum("deg")
_seg_sum_scaled = _make_seg_sum("input")
_seg_sum = _make_seg_sum("raw")


_BR = 1024
_GRID = NTAB // _BR

_row_spec = pl.BlockSpec((_BR, D), lambda i: (i, 0))
_rec_spec = pl.BlockSpec((_BR, 1), lambda i: (i, 0))
_w_spec = pl.BlockSpec((D, D), lambda i: (0, 0))
_b_spec = pl.BlockSpec((1, D), lambda i: (0, 0))


def _dot(a, w):
    return jax.lax.dot_general(a, w, (((1,), (0,)), ((), ())),
                               precision=jax.lax.Precision.HIGHEST,
                               preferred_element_type=F32)


def _mm_bias(x, w, b):
    def body(x_ref, w_ref, b_ref, o_ref):
        o_ref[...] = _dot(x_ref[...], w_ref[...]) + b_ref[...]

    return pl.pallas_call(
        body, grid=(_GRID,),
        in_specs=[_row_spec, _w_spec, _b_spec],
        out_specs=_row_spec,
        out_shape=jax.ShapeDtypeStruct((NTAB, D), F32),
    )(x, w, b.reshape(1, D))


def _relu_scale_mm(a, r, w, b):
    def body(a_ref, r_ref, w_ref, b_ref, o_ref):
        n = jnp.maximum(a_ref[...] * r_ref[...], 0.0)
        o_ref[...] = _dot(n, w_ref[...]) + b_ref[...]

    return pl.pallas_call(
        body, grid=(_GRID,),
        in_specs=[_row_spec, _rec_spec, _w_spec, _b_spec],
        out_specs=_row_spec,
        out_shape=jax.ShapeDtypeStruct((NTAB, D), F32),
    )(a, r.reshape(NTAB, 1), w, b.reshape(1, D))


def _final(a, r, wp1, bp1, wp2, bp2):
    def body(a_ref, r_ref, w1_ref, b1_ref, w2_ref, b2_ref, n_ref, p_ref):
        n = jnp.maximum(a_ref[...] * r_ref[...], 0.0)
        n_ref[...] = n
        h = _dot(n, w1_ref[...]) + b1_ref[...]
        h = jnp.where(h > 0.0, h, jnp.exp(jnp.minimum(h, 0.0)) - 1.0)
        p_ref[...] = _dot(h, w2_ref[...]) + b2_ref[...]

    return pl.pallas_call(
        body, grid=(_GRID,),
        in_specs=[_row_spec, _rec_spec, _w_spec, _b_spec, _w_spec, _b_spec],
        out_specs=[_row_spec, _row_spec],
        out_shape=[jax.ShapeDtypeStruct((NTAB, D), F32),
                   jax.ShapeDtypeStruct((NTAB, D), F32)],
    )(a, r.reshape(NTAB, 1), wp1, bp1.reshape(1, D),
      wp2, bp2.reshape(1, D))


def _pack_idx(i0, i1):
    both = jnp.stack([i0, i1])
    return both.reshape(NV * NS, TPR, IB)


def kernel(x0, x1, inc0, inc1, W0, b0, W1, b1, Wp1, bp1, Wp2, bp2):
    xs = jnp.concatenate(
        [x0, x1, jnp.zeros((NTAB - NV * NN, D), F32)], axis=0)
    v0, e0 = inc0[0], inc0[1]
    v1, e1 = inc1[0], inc1[1]
    pad_src = NV * NN + (jnp.arange(PAD, dtype=jnp.int32) % (NTAB - NV * NN))
    pad_dst = NN + (jnp.arange(PAD, dtype=jnp.int32) % TRASH_D)
    src_v = _pack_idx(jnp.concatenate([v0, pad_src]),
                      jnp.concatenate([v1 + NN, pad_src]))
    dst_e = _pack_idx(jnp.concatenate([e0, pad_dst]),
                      jnp.concatenate([e1, pad_dst]))
    src_e = _pack_idx(jnp.concatenate([e0, pad_src]),
                      jnp.concatenate([e1 + NN, pad_src]))
    dst_v = _pack_idx(jnp.concatenate([v0, pad_dst]),
                      jnp.concatenate([v1, pad_dst]))

    x_lin = _mm_bias(xs, W0, b0)
    y1, rec_e, rec_v = _seg_sum_deg(x_lin, src_v, dst_e)
    b1_raw, = _seg_sum(y1, src_e, dst_v)
    x2 = _relu_scale_mm(b1_raw, rec_v, W1, b1)
    y2, = _seg_sum_scaled(x2, src_v, dst_e, rec_e)
    b2_raw, = _seg_sum(y2, src_e, dst_v)
    n_out, p_out = _final(b2_raw, rec_v, Wp1, bp1, Wp2, bp2)
    return (n_out[:NN], n_out[NN:NV * NN], p_out[:NN], p_out[NN:NV * NN])
